# baseline (speedup 1.0000x reference)
import functools

import numpy as np

import jax
import jax.numpy as jnp
from jax import lax
from jax.experimental import pallas as pl
from jax.experimental.pallas import tpu as pltpu
from jax.experimental.pallas import tpu_sc as plsc

N = 10000
D = 128
DW = D // 2
E = 320000
NC = 2
NS = 16
NW = NC * NS
K = 125
NCHUNK = E // (NW * K)
BLK = 2
NBLK = NCHUNK // BLK
NGRP = NCHUNK // 4
RPT = 632
RPT_LAST = N - (NS - 1) * RPT
ZR = 40

_PLO = np.array([32 * (m // 16) + (m % 16) for m in range(DW)])
_PHI = _PLO + 16

_mesh = plsc.VectorSubcoreMesh(core_axis_name="c", subcore_axis_name="s")


@functools.partial(
    pl.kernel,
    out_type=jax.ShapeDtypeStruct((NC, N, D), jnp.float32),
    mesh=_mesh,
    compiler_params=pltpu.CompilerParams(use_tc_tiling_on_sc=False),
    scratch_types=[
        pltpu.VMEM_SHARED((N, D), jnp.float32),
        pltpu.VMEM((2, BLK, K), jnp.int32),
        pltpu.VMEM((2, BLK, K), jnp.int32),
        pltpu.VMEM((2, K, DW), jnp.int32),
        pltpu.VMEM((K, D), jnp.float32),
        pltpu.SemaphoreType.DMA,
        pltpu.SemaphoreType.DMA,
        pltpu.SemaphoreType.DMA,
        pltpu.SemaphoreType.DMA,
        pltpu.SemaphoreType.DMA,
        pltpu.SemaphoreType.DMA,
        pltpu.SemaphoreType.DMA,
    ],
)
def _sc_aggregate(xq_hbm, ei_hbm, out_hbm,
                  acc, sblk, dblk, rows, frows,
                  g0, g1, ssem, is0, is1, id0, id1):
    gsems = (g0, g1)
    isems = (is0, is1)
    idsems = (id0, id1)
    c = lax.axis_index("c")
    s = lax.axis_index("s")
    w = c * NS + s
    src_hbm = ei_hbm.at[0]
    dst_hbm = ei_hbm.at[1]

    pltpu.async_copy(src_hbm.at[w, 0], sblk.at[0], isems[0])
    pltpu.async_copy(dst_hbm.at[w, 0], dblk.at[0], idsems[0])

    zv = jnp.zeros((16,), jnp.float32)
    for i in range(ZR):
        for jj in range(D // 16):
            frows[i, pl.ds(jj * 16, 16)] = zv

    zbase = s * RPT

    @pl.when(s < NS - 1)
    def _():
        for r in range(15):
            pltpu.async_copy(frows.at[pl.ds(0, ZR)],
                             acc.at[pl.ds(zbase + r * ZR, ZR)], ssem)
        pltpu.async_copy(frows.at[pl.ds(0, 32)],
                         acc.at[pl.ds(zbase + 15 * ZR, 32)], ssem)
        for r in range(15):
            pltpu.make_async_copy(frows.at[pl.ds(0, ZR)],
                                  acc.at[pl.ds(zbase + r * ZR, ZR)],
                                  ssem).wait()
        pltpu.make_async_copy(frows.at[pl.ds(0, 32)],
                              acc.at[pl.ds(zbase + 15 * ZR, 32)],
                              ssem).wait()

    @pl.when(s == NS - 1)
    def _():
        for r in range(13):
            pltpu.async_copy(frows.at[pl.ds(0, ZR)],
                             acc.at[pl.ds(zbase + r * ZR, ZR)], ssem)
        for r in range(13):
            pltpu.make_async_copy(frows.at[pl.ds(0, ZR)],
                                  acc.at[pl.ds(zbase + r * ZR, ZR)],
                                  ssem).wait()

    pltpu.make_async_copy(src_hbm.at[w, 0], sblk.at[0], isems[0]).wait()
    pltpu.make_async_copy(dst_hbm.at[w, 0], dblk.at[0], idsems[0]).wait()

    plsc.subcore_barrier()

    pltpu.async_copy(xq_hbm.at[sblk.at[0, 0]], rows.at[0], gsems[0])

    hi_mask = jnp.int32(-65536)

    def _convert(slot):
        @plsc.parallel_loop(0, K, unroll=25)
        def conv_row(rw):
            for g2 in range(DW // 16):
                v = rows[slot, rw, pl.ds(g2 * 16, 16)]
                lo = lax.bitcast_convert_type(lax.shift_left(v, 16),
                                              jnp.float32)
                hi = lax.bitcast_convert_type(v & hi_mask, jnp.float32)
                frows[rw, pl.ds(g2 * 32, 16)] = lo
                frows[rw, pl.ds(g2 * 32 + 16, 16)] = hi

    def body(g, carry):
        for u in range(4):
            ru, rn = u % 2, (u + 1) % 2
            p, r = u // 2, u % 2

            pltpu.make_async_copy(xq_hbm.at[sblk.at[p, r]], rows.at[ru],
                                  gsems[ru]).wait()

            np_, nr = ((u + 1) // 2) % 2, (u + 1) % 2
            if u == 1:
                blk = 2 * g + 1
                pltpu.make_async_copy(src_hbm.at[w, blk], sblk.at[1],
                                      isems[1]).wait()
                pltpu.make_async_copy(dst_hbm.at[w, blk], dblk.at[1],
                                      idsems[1]).wait()
            if u == 3:
                @pl.when(g < NGRP - 1)
                def _():
                    blk = 2 * g + 2
                    pltpu.make_async_copy(src_hbm.at[w, blk], sblk.at[0],
                                          isems[0]).wait()
                    pltpu.make_async_copy(dst_hbm.at[w, blk], dblk.at[0],
                                          idsems[0]).wait()
                    pltpu.async_copy(xq_hbm.at[sblk.at[0, 0]], rows.at[rn],
                                     gsems[rn])
            else:
                pltpu.async_copy(xq_hbm.at[sblk.at[np_, nr]], rows.at[rn],
                                 gsems[rn])

            pp, rr = ((u - 1) % 4) // 2, (u - 1) % 2
            if u == 0:
                @pl.when(g >= 1)
                def _():
                    pltpu.make_async_copy(frows, acc.at[dblk.at[1, 1]],
                                          ssem).wait()
            else:
                pltpu.make_async_copy(frows, acc.at[dblk.at[pp, rr]],
                                      ssem).wait()

            _convert(ru)

            pltpu.async_copy(frows, acc.at[dblk.at[p, r]], ssem, add=True)

            if u == 0:
                blk = 2 * g + 1
                pltpu.async_copy(src_hbm.at[w, blk], sblk.at[1], isems[1])
                pltpu.async_copy(dst_hbm.at[w, blk], dblk.at[1], idsems[1])
            if u == 2:
                @pl.when(g < NGRP - 1)
                def _():
                    blk = 2 * g + 2
                    pltpu.async_copy(src_hbm.at[w, blk], sblk.at[0], isems[0])
                    pltpu.async_copy(dst_hbm.at[w, blk], dblk.at[0],
                                     idsems[0])
        return carry

    lax.fori_loop(0, NGRP, body, 0)

    pltpu.make_async_copy(frows, acc.at[dblk.at[1, 1]], ssem).wait()

    plsc.subcore_barrier()

    @pl.when(s < NS - 1)
    def _():
        pltpu.sync_copy(acc.at[pl.ds(s * RPT, RPT)],
                        out_hbm.at[c, pl.ds(s * RPT, RPT)])

    @pl.when(s == NS - 1)
    def _():
        pltpu.sync_copy(acc.at[pl.ds((NS - 1) * RPT, RPT_LAST)],
                        out_hbm.at[c, pl.ds((NS - 1) * RPT, RPT_LAST)])


BN = 1000


def _tc_body(x_ref, p_ref, w_ref, b_ref, o_ref):
    h = x_ref[...] + p_ref[0] + p_ref[1]
    o_ref[...] = lax.dot_general(
        h, w_ref[...], (((1,), (1,)), ((), ())),
        preferred_element_type=jnp.float32) + b_ref[...]


def _tc_combine(x, parts, W, b2):
    return pl.pallas_call(
        _tc_body,
        grid=(N // BN,),
        in_specs=[
            pl.BlockSpec((BN, D), lambda i: (i, 0)),
            pl.BlockSpec((NC, BN, D), lambda i: (0, i, 0)),
            pl.BlockSpec((D, D), lambda i: (0, 0)),
            pl.BlockSpec((1, D), lambda i: (0, 0)),
        ],
        out_specs=pl.BlockSpec((BN, D), lambda i: (i, 0)),
        out_shape=jax.ShapeDtypeStruct((N, D), jnp.float32),
    )(x, parts, W, b2)


def kernel(x, edge_index, W, b):
    ei = edge_index.astype(jnp.int32).reshape(2, NW, NBLK, BLK, K)
    xq = x.astype(jnp.bfloat16)
    xi = jnp.stack([xq[:, _PLO], xq[:, _PHI]], axis=-1)
    xq32 = lax.bitcast_convert_type(xi, jnp.int32)
    parts = _sc_aggregate(xq32, ei)
    return _tc_combine(x, parts, W, b.reshape(1, D))


# speedup vs baseline: 1.0933x; 1.0933x over previous
import functools

import jax
import jax.numpy as jnp
from jax import lax
from jax.experimental import pallas as pl
from jax.experimental.pallas import tpu as pltpu
from jax.experimental.pallas import tpu_sc as plsc

N = 10000
D = 128
DW = D // 2
E = 320000
NC = 2
NS = 16
NW = NC * NS
K = 125
NCHUNK = E // (NW * K)
BLK = 2
NBLK = NCHUNK // BLK
NGRP = NCHUNK // 4
RPT = 632
RPT_LAST = N - (NS - 1) * RPT
ZR = 40

_mesh = plsc.VectorSubcoreMesh(core_axis_name="c", subcore_axis_name="s")


@functools.partial(
    pl.kernel,
    out_type=jax.ShapeDtypeStruct((NC, 2, N, DW), jnp.float32),
    mesh=_mesh,
    compiler_params=pltpu.CompilerParams(use_tc_tiling_on_sc=False),
    scratch_types=[
        pltpu.VMEM_SHARED((N, DW), jnp.float32),
        pltpu.VMEM_SHARED((N, DW), jnp.float32),
        pltpu.VMEM((2, BLK, K), jnp.int32),
        pltpu.VMEM((2, BLK, K), jnp.int32),
        pltpu.VMEM((2, K, DW), jnp.float32),
        pltpu.VMEM((K, DW), jnp.float32),
        pltpu.SemaphoreType.DMA,
        pltpu.SemaphoreType.DMA,
        pltpu.SemaphoreType.DMA,
        pltpu.SemaphoreType.DMA,
        pltpu.SemaphoreType.DMA,
        pltpu.SemaphoreType.DMA,
        pltpu.SemaphoreType.DMA,
        pltpu.SemaphoreType.DMA,
        pltpu.SemaphoreType.DMA,
    ],
)
def _sc_aggregate(xq_hbm, ei_hbm, out_hbm,
                  acc_lo, acc_hi, sblk, dblk, rows, flo,
                  g0, g1, slo, sh0, sh1, is0, is1, id0, id1):
    gsems = (g0, g1)
    shsems = (sh0, sh1)
    isems = (is0, is1)
    idsems = (id0, id1)
    accs = (acc_lo, acc_hi)
    c = lax.axis_index("c")
    s = lax.axis_index("s")
    w = c * NS + s
    src_hbm = ei_hbm.at[0]
    dst_hbm = ei_hbm.at[1]

    pltpu.async_copy(src_hbm.at[w, 0], sblk.at[0], isems[0])
    pltpu.async_copy(dst_hbm.at[w, 0], dblk.at[0], idsems[0])

    zv = jnp.zeros((16,), jnp.float32)
    for i in range(ZR):
        for jj in range(DW // 16):
            flo[i, pl.ds(jj * 16, 16)] = zv

    zbase = s * RPT

    @pl.when(s < NS - 1)
    def _():
        for a in range(2):
            for r in range(15):
                pltpu.async_copy(flo.at[pl.ds(0, ZR)],
                                 accs[a].at[pl.ds(zbase + r * ZR, ZR)],
                                 shsems[a])
            pltpu.async_copy(flo.at[pl.ds(0, 32)],
                             accs[a].at[pl.ds(zbase + 15 * ZR, 32)],
                             shsems[a])
        for a in range(2):
            for r in range(15):
                pltpu.make_async_copy(flo.at[pl.ds(0, ZR)],
                                      accs[a].at[pl.ds(zbase + r * ZR, ZR)],
                                      shsems[a]).wait()
            pltpu.make_async_copy(flo.at[pl.ds(0, 32)],
                                  accs[a].at[pl.ds(zbase + 15 * ZR, 32)],
                                  shsems[a]).wait()

    @pl.when(s == NS - 1)
    def _():
        for a in range(2):
            for r in range(13):
                pltpu.async_copy(flo.at[pl.ds(0, ZR)],
                                 accs[a].at[pl.ds(zbase + r * ZR, ZR)],
                                 shsems[a])
        for a in range(2):
            for r in range(13):
                pltpu.make_async_copy(flo.at[pl.ds(0, ZR)],
                                      accs[a].at[pl.ds(zbase + r * ZR, ZR)],
                                      shsems[a]).wait()

    pltpu.make_async_copy(src_hbm.at[w, 0], sblk.at[0], isems[0]).wait()
    pltpu.make_async_copy(dst_hbm.at[w, 0], dblk.at[0], idsems[0]).wait()

    plsc.subcore_barrier()

    pltpu.async_copy(xq_hbm.at[sblk.at[0, 0]], rows.at[0], gsems[0])

    def _shift_lo(slot):
        @plsc.parallel_loop(0, K, unroll=5)
        def conv_row(rw):
            for g2 in range(DW // 16):
                v = lax.bitcast_convert_type(
                    rows[slot, rw, pl.ds(g2 * 16, 16)], jnp.int32)
                flo[rw, pl.ds(g2 * 16, 16)] = lax.bitcast_convert_type(
                    lax.shift_left(v, 16), jnp.float32)

    def body(g, carry):
        for u in range(4):
            ru, rn = u % 2, (u + 1) % 2
            p, r = u // 2, u % 2
            pp, rr = ((u - 1) % 4) // 2, (u - 1) % 2

            pltpu.make_async_copy(xq_hbm.at[sblk.at[p, r]], rows.at[ru],
                                  gsems[ru]).wait()

            if u == 0:
                @pl.when(g >= 1)
                def _():
                    pltpu.make_async_copy(rows.at[rn],
                                          acc_hi.at[dblk.at[1, 1]],
                                          shsems[rn]).wait()
            else:
                pltpu.make_async_copy(rows.at[rn],
                                      acc_hi.at[dblk.at[pp, rr]],
                                      shsems[rn]).wait()

            np_, nr = ((u + 1) // 2) % 2, (u + 1) % 2
            if u == 1:
                blk = 2 * g + 1
                pltpu.make_async_copy(src_hbm.at[w, blk], sblk.at[1],
                                      isems[1]).wait()
                pltpu.make_async_copy(dst_hbm.at[w, blk], dblk.at[1],
                                      idsems[1]).wait()
            if u == 3:
                @pl.when(g < NGRP - 1)
                def _():
                    blk = 2 * g + 2
                    pltpu.make_async_copy(src_hbm.at[w, blk], sblk.at[0],
                                          isems[0]).wait()
                    pltpu.make_async_copy(dst_hbm.at[w, blk], dblk.at[0],
                                          idsems[0]).wait()
                    pltpu.async_copy(xq_hbm.at[sblk.at[0, 0]], rows.at[rn],
                                     gsems[rn])
            else:
                pltpu.async_copy(xq_hbm.at[sblk.at[np_, nr]], rows.at[rn],
                                 gsems[rn])

            if u == 0:
                @pl.when(g >= 1)
                def _():
                    pltpu.make_async_copy(flo, acc_lo.at[dblk.at[1, 1]],
                                          slo).wait()
            else:
                pltpu.make_async_copy(flo, acc_lo.at[dblk.at[pp, rr]],
                                      slo).wait()

            _shift_lo(ru)

            pltpu.async_copy(rows.at[ru], acc_hi.at[dblk.at[p, r]],
                             shsems[ru], add=True)
            pltpu.async_copy(flo, acc_lo.at[dblk.at[p, r]], slo, add=True)

            if u == 0:
                blk = 2 * g + 1
                pltpu.async_copy(src_hbm.at[w, blk], sblk.at[1], isems[1])
                pltpu.async_copy(dst_hbm.at[w, blk], dblk.at[1], idsems[1])
            if u == 2:
                @pl.when(g < NGRP - 1)
                def _():
                    blk = 2 * g + 2
                    pltpu.async_copy(src_hbm.at[w, blk], sblk.at[0], isems[0])
                    pltpu.async_copy(dst_hbm.at[w, blk], dblk.at[0],
                                     idsems[0])
        return carry

    lax.fori_loop(0, NGRP, body, 0)

    pltpu.make_async_copy(rows.at[1], acc_hi.at[dblk.at[1, 1]],
                          shsems[1]).wait()
    pltpu.make_async_copy(flo, acc_lo.at[dblk.at[1, 1]], slo).wait()

    plsc.subcore_barrier()

    @pl.when(s < NS - 1)
    def _():
        for a in range(2):
            pltpu.sync_copy(accs[a].at[pl.ds(s * RPT, RPT)],
                            out_hbm.at[c, a, pl.ds(s * RPT, RPT)])

    @pl.when(s == NS - 1)
    def _():
        for a in range(2):
            pltpu.sync_copy(accs[a].at[pl.ds((NS - 1) * RPT, RPT_LAST)],
                            out_hbm.at[c, a, pl.ds((NS - 1) * RPT, RPT_LAST)])


BN = 1000


def _tc_body(x_ref, p_ref, w_ref, b_ref, o_ref):
    lo = p_ref[0, 0] + p_ref[1, 0]
    hi = p_ref[0, 1] + p_ref[1, 1]
    h = x_ref[...] + jnp.concatenate([lo, hi], axis=1)
    o_ref[...] = lax.dot_general(
        h, w_ref[...], (((1,), (1,)), ((), ())),
        preferred_element_type=jnp.float32) + b_ref[...]


def _tc_combine(x, parts, W, b2):
    return pl.pallas_call(
        _tc_body,
        grid=(N // BN,),
        in_specs=[
            pl.BlockSpec((BN, D), lambda i: (i, 0)),
            pl.BlockSpec((NC, 2, BN, DW), lambda i: (0, 0, i, 0)),
            pl.BlockSpec((D, D), lambda i: (0, 0)),
            pl.BlockSpec((1, D), lambda i: (0, 0)),
        ],
        out_specs=pl.BlockSpec((BN, D), lambda i: (i, 0)),
        out_shape=jax.ShapeDtypeStruct((N, D), jnp.float32),
    )(x, parts, W, b2)


def kernel(x, edge_index, W, b):
    ei = edge_index.astype(jnp.int32).reshape(2, NW, NBLK, BLK, K)
    xq = x.astype(jnp.bfloat16)
    xi = jnp.stack([xq[:, :DW], xq[:, DW:]], axis=-1)
    xqf = lax.bitcast_convert_type(
        lax.bitcast_convert_type(xi, jnp.int32), jnp.float32)
    parts = _sc_aggregate(xqf, ei)
    return _tc_combine(x, parts, W, b.reshape(1, D))


# speedup vs baseline: 1.0958x; 1.0023x over previous
import functools

import jax
import jax.numpy as jnp
from jax import lax
from jax.experimental import pallas as pl
from jax.experimental.pallas import tpu as pltpu
from jax.experimental.pallas import tpu_sc as plsc

N = 10000
D = 128
DW = D // 2
E = 320000
NC = 2
NS = 16
NW = NC * NS
K = 125
NCHUNK = E // (NW * K)
BLK = 2
NBLK = NCHUNK // BLK
NGRP = NCHUNK // 4
RPT = 632
RPT_LAST = N - (NS - 1) * RPT
ZR = 40

_mesh = plsc.VectorSubcoreMesh(core_axis_name="c", subcore_axis_name="s")


@functools.partial(
    pl.kernel,
    out_type=jax.ShapeDtypeStruct((NC, 2, N, DW), jnp.float32),
    mesh=_mesh,
    compiler_params=pltpu.CompilerParams(use_tc_tiling_on_sc=False),
    scratch_types=[
        pltpu.VMEM_SHARED((N, DW), jnp.float32),
        pltpu.VMEM_SHARED((N, DW), jnp.float32),
        pltpu.VMEM((2, BLK, K), jnp.int32),
        pltpu.VMEM((2, BLK, K), jnp.int32),
        pltpu.VMEM((2, K, DW), jnp.float32),
        pltpu.VMEM((K, DW), jnp.float32),
        pltpu.SemaphoreType.DMA,
        pltpu.SemaphoreType.DMA,
        pltpu.SemaphoreType.DMA,
        pltpu.SemaphoreType.DMA,
        pltpu.SemaphoreType.DMA,
        pltpu.SemaphoreType.DMA,
        pltpu.SemaphoreType.DMA,
        pltpu.SemaphoreType.DMA,
        pltpu.SemaphoreType.DMA,
    ],
)
def _sc_aggregate(xq_hbm, ei_hbm, out_hbm,
                  acc_lo, acc_hi, sblk, dblk, rows, flo,
                  g0, g1, slo, sh0, sh1, is0, is1, id0, id1):
    gsems = (g0, g1)
    shsems = (sh0, sh1)
    isems = (is0, is1)
    idsems = (id0, id1)
    accs = (acc_lo, acc_hi)
    c = lax.axis_index("c")
    s = lax.axis_index("s")
    w = c * NS + s
    src_hbm = ei_hbm.at[0]
    dst_hbm = ei_hbm.at[1]

    pltpu.async_copy(src_hbm.at[w, 0], sblk.at[0], isems[0])
    pltpu.async_copy(dst_hbm.at[w, 0], dblk.at[0], idsems[0])

    zv = jnp.zeros((16,), jnp.float32)
    for i in range(ZR):
        for jj in range(DW // 16):
            flo[i, pl.ds(jj * 16, 16)] = zv

    zbase = s * RPT

    @pl.when(s < NS - 1)
    def _():
        for a in range(2):
            for r in range(15):
                pltpu.async_copy(flo.at[pl.ds(0, ZR)],
                                 accs[a].at[pl.ds(zbase + r * ZR, ZR)],
                                 shsems[a])
            pltpu.async_copy(flo.at[pl.ds(0, 32)],
                             accs[a].at[pl.ds(zbase + 15 * ZR, 32)],
                             shsems[a])
        for a in range(2):
            for r in range(15):
                pltpu.make_async_copy(flo.at[pl.ds(0, ZR)],
                                      accs[a].at[pl.ds(zbase + r * ZR, ZR)],
                                      shsems[a]).wait()
            pltpu.make_async_copy(flo.at[pl.ds(0, 32)],
                                  accs[a].at[pl.ds(zbase + 15 * ZR, 32)],
                                  shsems[a]).wait()

    @pl.when(s == NS - 1)
    def _():
        for a in range(2):
            for r in range(13):
                pltpu.async_copy(flo.at[pl.ds(0, ZR)],
                                 accs[a].at[pl.ds(zbase + r * ZR, ZR)],
                                 shsems[a])
        for a in range(2):
            for r in range(13):
                pltpu.make_async_copy(flo.at[pl.ds(0, ZR)],
                                      accs[a].at[pl.ds(zbase + r * ZR, ZR)],
                                      shsems[a]).wait()

    pltpu.make_async_copy(src_hbm.at[w, 0], sblk.at[0], isems[0]).wait()
    pltpu.make_async_copy(dst_hbm.at[w, 0], dblk.at[0], idsems[0]).wait()

    plsc.subcore_barrier()

    pltpu.async_copy(xq_hbm.at[sblk.at[0, 0]], rows.at[0], gsems[0])

    def _shift_lo(slot):
        @plsc.parallel_loop(0, K, unroll=5)
        def conv_row(rw):
            for g2 in range(DW // 16):
                v = lax.bitcast_convert_type(
                    rows[slot, rw, pl.ds(g2 * 16, 16)], jnp.int32)
                flo[rw, pl.ds(g2 * 16, 16)] = lax.bitcast_convert_type(
                    lax.shift_left(v, 16), jnp.float32)

    def body(g, carry):
        for u in range(4):
            ru, rn = u % 2, (u + 1) % 2
            p, r = u // 2, u % 2
            pp, rr = ((u - 1) % 4) // 2, (u - 1) % 2

            pltpu.make_async_copy(xq_hbm.at[sblk.at[p, r]], rows.at[ru],
                                  gsems[ru]).wait()
            pltpu.async_copy(rows.at[ru], acc_hi.at[dblk.at[p, r]],
                             shsems[ru], add=True)

            if u == 0:
                @pl.when(g >= 1)
                def _():
                    pltpu.make_async_copy(rows.at[rn],
                                          acc_hi.at[dblk.at[1, 1]],
                                          shsems[rn]).wait()
            else:
                pltpu.make_async_copy(rows.at[rn],
                                      acc_hi.at[dblk.at[pp, rr]],
                                      shsems[rn]).wait()

            np_, nr = ((u + 1) // 2) % 2, (u + 1) % 2
            if u == 1:
                blk = 2 * g + 1
                pltpu.make_async_copy(src_hbm.at[w, blk], sblk.at[1],
                                      isems[1]).wait()
                pltpu.make_async_copy(dst_hbm.at[w, blk], dblk.at[1],
                                      idsems[1]).wait()
            if u == 3:
                @pl.when(g < NGRP - 1)
                def _():
                    blk = 2 * g + 2
                    pltpu.make_async_copy(src_hbm.at[w, blk], sblk.at[0],
                                          isems[0]).wait()
                    pltpu.make_async_copy(dst_hbm.at[w, blk], dblk.at[0],
                                          idsems[0]).wait()
                    pltpu.async_copy(xq_hbm.at[sblk.at[0, 0]], rows.at[rn],
                                     gsems[rn])
            else:
                pltpu.async_copy(xq_hbm.at[sblk.at[np_, nr]], rows.at[rn],
                                 gsems[rn])

            if u == 0:
                @pl.when(g >= 1)
                def _():
                    pltpu.make_async_copy(flo, acc_lo.at[dblk.at[1, 1]],
                                          slo).wait()
            else:
                pltpu.make_async_copy(flo, acc_lo.at[dblk.at[pp, rr]],
                                      slo).wait()

            _shift_lo(ru)

            pltpu.async_copy(flo, acc_lo.at[dblk.at[p, r]], slo, add=True)

            if u == 0:
                blk = 2 * g + 1
                pltpu.async_copy(src_hbm.at[w, blk], sblk.at[1], isems[1])
                pltpu.async_copy(dst_hbm.at[w, blk], dblk.at[1], idsems[1])
            if u == 2:
                @pl.when(g < NGRP - 1)
                def _():
                    blk = 2 * g + 2
                    pltpu.async_copy(src_hbm.at[w, blk], sblk.at[0], isems[0])
                    pltpu.async_copy(dst_hbm.at[w, blk], dblk.at[0],
                                     idsems[0])
        return carry

    lax.fori_loop(0, NGRP, body, 0)

    pltpu.make_async_copy(rows.at[1], acc_hi.at[dblk.at[1, 1]],
                          shsems[1]).wait()
    pltpu.make_async_copy(flo, acc_lo.at[dblk.at[1, 1]], slo).wait()

    plsc.subcore_barrier()

    @pl.when(s < NS - 1)
    def _():
        for a in range(2):
            pltpu.sync_copy(accs[a].at[pl.ds(s * RPT, RPT)],
                            out_hbm.at[c, a, pl.ds(s * RPT, RPT)])

    @pl.when(s == NS - 1)
    def _():
        for a in range(2):
            pltpu.sync_copy(accs[a].at[pl.ds((NS - 1) * RPT, RPT_LAST)],
                            out_hbm.at[c, a, pl.ds((NS - 1) * RPT, RPT_LAST)])


BN = 1000


def _tc_body(x_ref, p_ref, w_ref, b_ref, o_ref):
    lo = p_ref[0, 0] + p_ref[1, 0]
    hi = p_ref[0, 1] + p_ref[1, 1]
    h = x_ref[...] + jnp.concatenate([lo, hi], axis=1)
    o_ref[...] = lax.dot_general(
        h, w_ref[...], (((1,), (1,)), ((), ())),
        preferred_element_type=jnp.float32) + b_ref[...]


def _tc_combine(x, parts, W, b2):
    return pl.pallas_call(
        _tc_body,
        grid=(N // BN,),
        in_specs=[
            pl.BlockSpec((BN, D), lambda i: (i, 0)),
            pl.BlockSpec((NC, 2, BN, DW), lambda i: (0, 0, i, 0)),
            pl.BlockSpec((D, D), lambda i: (0, 0)),
            pl.BlockSpec((1, D), lambda i: (0, 0)),
        ],
        out_specs=pl.BlockSpec((BN, D), lambda i: (i, 0)),
        out_shape=jax.ShapeDtypeStruct((N, D), jnp.float32),
    )(x, parts, W, b2)


def kernel(x, edge_index, W, b):
    ei = edge_index.astype(jnp.int32).reshape(2, NW, NBLK, BLK, K)
    xq = x.astype(jnp.bfloat16)
    xi = jnp.stack([xq[:, :DW], xq[:, DW:]], axis=-1)
    xqf = lax.bitcast_convert_type(
        lax.bitcast_convert_type(xi, jnp.int32), jnp.float32)
    parts = _sc_aggregate(xqf, ei)
    return _tc_combine(x, parts, W, b.reshape(1, D))
